# Initial kernel scaffold; baseline (speedup 1.0000x reference)
import jax
import jax.numpy as jnp
from jax.experimental import pallas as pl


def kernel(node_features, rel_emb, W1a, b1a, W2a, b2a, W1b, b1b, W2b, b2b, edge_batch_idx, edge_neighbor, edge_rel, batch_node_ids):
    raise NotImplementedError("write your pallas kernel here")


# speedup vs baseline: 10.2884x; 10.2884x over previous
import jax
import jax.numpy as jnp
from jax import lax
from jax.experimental import pallas as pl
from jax.experimental.pallas import tpu as pltpu
from jax.experimental.pallas import tpu_sc as plsc

D = 128
DH = 64
HALF = 64
B = 4096
E = 262144
NREL = 9
TEMP = 0.07

NC = 1
NS = 16
EPT = E // NS
K = 128
NCHUNK = EPT // K
NBLK = 16
CPB = NCHUNK // NBLK
CNT_T = B * NREL // NS

_f32 = jnp.float32
_i32 = jnp.int32


def _sc_body(nf_hbm, cmb_hbm, enb_hbm, bni_hbm, out_acc, out_cnt, out_fb,
             nbr1d, cmb1d, sidx2d, cidx2d, bid_c, rows_v, ones1d, cst1d,
             acc, acc_cnt, sem):
    s = lax.axis_index("s")
    ebase = s * EPT

    zero16 = jnp.zeros((16,), _f32)

    ones16 = jnp.ones((16,), _f32)

    def _zrow(i, _):
        for cc in range(8):
            rows_v[i, pl.ds(cc * 16, 16)] = zero16
        return 0
    lax.fori_loop(0, K, _zrow, 0)
    for cc in range(8):
        ones1d[pl.ds(cc * 16, 16)] = ones16

    def _zcnt(i, _):
        cst1d[pl.ds(i * 16, 16)] = zero16
        return 0
    lax.fori_loop(0, CNT_T // 16, _zcnt, 0)
    for k in range(4):
        acc_r = s * 512 + k * K
        pltpu.sync_copy(rows_v, acc.at[pl.ds(acc_r, K)])
    pltpu.sync_copy(cst1d, acc_cnt.at[pl.ds(s * CNT_T, CNT_T)])
    plsc.subcore_barrier()

    def _block(bi, _):
        boff = ebase + bi * (CPB * K)
        pltpu.sync_copy(cmb_hbm.at[pl.ds(boff, CPB * K)], cmb1d)
        pltpu.sync_copy(enb_hbm.at[pl.ds(boff, CPB * K)], nbr1d)

        def _chunk(j, _):
            base = j * K
            for cc in range(8):
                off = base + cc * 16
                cmb = cmb1d[pl.ds(off, 16)]
                bat = lax.shift_right_logical(cmb, 4)
                rel = cmb & _i32(15)
                v2 = (rel >= 1) & (rel <= 6)
                sidx2d[j, pl.ds(cc * 16, 16)] = bat + jnp.where(
                    v2, _i32(B), _i32(0))
                cidx2d[j, pl.ds(cc * 16, 16)] = bat * _i32(NREL) + rel
            pltpu.async_copy(nf_hbm.at[nbr1d.at[pl.ds(base, K)]], rows_v,
                             sem).wait()
            pltpu.sync_copy(rows_v, acc.at[sidx2d.at[j]], add=True)
            pltpu.sync_copy(ones1d, acc_cnt.at[cidx2d.at[j]], add=True)
            return 0
        lax.fori_loop(0, CPB, _chunk, 0)
        return 0
    lax.fori_loop(0, NBLK, _block, 0)

    for k in range(2):
        fbase = s * 256 + k * K
        pltpu.sync_copy(bni_hbm.at[pl.ds(fbase, K)], bid_c)
        pltpu.async_copy(nf_hbm.at[bid_c], rows_v, sem).wait()
        pltpu.sync_copy(rows_v, out_fb.at[pl.ds(fbase, K)])

    plsc.subcore_barrier()
    for k in range(4):
        acc_r = s * 512 + k * K
        pltpu.sync_copy(acc.at[pl.ds(acc_r, K)], rows_v)
        pltpu.sync_copy(rows_v, out_acc.at[pl.ds(acc_r, K)])
    pltpu.sync_copy(acc_cnt.at[pl.ds(s * CNT_T, CNT_T)], cst1d)
    pltpu.sync_copy(cst1d, out_cnt.at[pl.ds(s * CNT_T, CNT_T)])


_sc_agg = pl.kernel(
    _sc_body,
    out_type=(
        jax.ShapeDtypeStruct((2 * B, D), _f32),
        jax.ShapeDtypeStruct((B * NREL,), _f32),
        jax.ShapeDtypeStruct((B, D), _f32),
    ),
    mesh=plsc.VectorSubcoreMesh(
        core_axis_name="c", subcore_axis_name="s",
        num_cores=NC, num_subcores=NS),
    scratch_types=(
        pltpu.VMEM((CPB * K,), _i32),
        pltpu.VMEM((CPB * K,), _i32),
        pltpu.VMEM((CPB, K), _i32),
        pltpu.VMEM((CPB, K), _i32),
        pltpu.VMEM((K,), _i32),
        pltpu.VMEM((K, D), _f32),
        pltpu.VMEM((K,), _f32),
        pltpu.VMEM((CNT_T,), _f32),
        pltpu.VMEM_SHARED((2 * B, D), _f32),
        pltpu.VMEM_SHARED((B * NREL,), _f32),
        pltpu.SemaphoreType.DMA,
    ),
)


def _tc_body(accp, cntp, fb, relp, w1a, b1a, w2a, b2a, w1b, b1b, w2b, b2b,
             out):
    cnt = cntp[...]
    col = lax.broadcasted_iota(_i32, (B, NREL), 1)
    m1 = (col == 0) | (col == 7) | (col == 8)
    cnt1 = jnp.where(m1, cnt, 0.0)
    cnt2 = jnp.where(m1, 0.0, cnt)
    n1 = jnp.sum(cnt1, axis=1, keepdims=True)
    n2 = jnp.sum(cnt2, axis=1, keepdims=True)

    rel = relp[...]
    s1 = accp[0:B] + jnp.dot(cnt1, rel, preferred_element_type=_f32)
    s2 = accp[B:2 * B] + jnp.dot(cnt2, rel, preferred_element_type=_f32)
    fbv = fb[...]
    a1 = jnp.where(n1 > 0, s1 / jnp.maximum(n1, 1.0), fbv)
    a2 = jnp.where(n2 > 0, s2 / jnp.maximum(n2, 1.0), fbv)

    def proj(x, w1, b1, w2, b2):
        h = jnp.maximum(jnp.dot(x, w1[...], preferred_element_type=_f32)
                        + b1[...], 0.0)
        return jnp.dot(h, w2[...], preferred_element_type=_f32) + b2[...]

    def l2n(z):
        nrm = jnp.sqrt(jnp.sum(z * z, axis=1, keepdims=True))
        return z / jnp.maximum(nrm, 1e-12)

    z1 = l2n(proj(a1, w1a, b1a, w2a, b2a))
    z2 = l2n(proj(a2, w1b, b1b, w2b, b2b))

    inv_t = _f32(1.0 / TEMP)
    CB = 512
    m = jnp.full((B, 1), -1e30, _f32)
    ssum = jnp.zeros((B, 1), _f32)
    col_acc = _f32(0.0)
    diag_acc = _f32(0.0)
    for j in range(B // CB):
        z2b = z2[j * CB:(j + 1) * CB]
        simb = lax.dot_general(z1, z2b, (((1,), (1,)), ((), ())),
                               preferred_element_type=_f32) * inv_t
        bm = jnp.max(simb, axis=1, keepdims=True)
        mn = jnp.maximum(m, bm)
        ssum = ssum * jnp.exp(m - mn) + jnp.sum(
            jnp.exp(simb - mn), axis=1, keepdims=True)
        m = mn
        cm = jnp.max(simb, axis=0, keepdims=True)
        col_acc = col_acc + jnp.sum(
            cm + jnp.log(jnp.sum(jnp.exp(simb - cm), axis=0, keepdims=True)))
        ri = lax.broadcasted_iota(_i32, (B, CB), 0)
        ci = lax.broadcasted_iota(_i32, (B, CB), 1) + j * CB
        diag_acc = diag_acc + jnp.sum(jnp.where(ri == ci, simb, 0.0))
    row_acc = jnp.sum(m + jnp.log(ssum))
    loss = -diag_acc / B + (row_acc + col_acc) / (2.0 * B)
    out[...] = loss.reshape(1, 1)


_tc_loss = pl.pallas_call(
    _tc_body,
    out_shape=jax.ShapeDtypeStruct((1, 1), _f32),
)


def kernel(node_features, rel_emb, W1a, b1a, W2a, b2a, W1b, b1b, W2b, b2b,
           edge_batch_idx, edge_neighbor, edge_rel, batch_node_ids):
    ebi = edge_batch_idx.astype(_i32)
    enb = edge_neighbor.astype(_i32)
    erl = edge_rel.astype(_i32)
    bni = batch_node_ids.astype(_i32)
    cmb = ebi * 16 + erl
    out_acc, out_cnt, fbv = _sc_agg(node_features, cmb, enb, bni)
    cnt = out_cnt.reshape(B, NREL)
    loss = _tc_loss(out_acc, cnt, fbv, rel_emb,
                    W1a, b1a.reshape(1, D), W2a, b2a.reshape(1, HALF),
                    W1b, b1b.reshape(1, D), W2b, b2b.reshape(1, HALF))
    return loss.reshape(())


# speedup vs baseline: 13.4760x; 1.3098x over previous
import jax
import jax.numpy as jnp
from jax import lax
from jax.experimental import pallas as pl
from jax.experimental.pallas import tpu as pltpu
from jax.experimental.pallas import tpu_sc as plsc

D = 128
DH = 64
HALF = 64
B = 4096
E = 262144
NREL = 9
TEMP = 0.07

NC = 1
NS = 16
EPT = E // NS
K = 128
NCHUNK = EPT // K
NBLK = 16
CPB = NCHUNK // NBLK
CNT_T = B * NREL // NS

_f32 = jnp.float32
_i32 = jnp.int32


def _sc_body(nf_hbm, cmb_hbm, enb_hbm, bni_hbm, out_acc, out_cnt, out_fb,
             nbr1d, cmb1d, sidx2d, cidx2d, bid_c, rows_v, rows_w, ones1d,
             cst1d, acc, acc_cnt, sem, sem2):
    s = lax.axis_index("s")
    ebase = s * EPT

    zero16 = jnp.zeros((16,), _f32)

    ones16 = jnp.ones((16,), _f32)

    def _zrow(i, _):
        for cc in range(8):
            rows_v[i, pl.ds(cc * 16, 16)] = zero16
        return 0
    lax.fori_loop(0, K, _zrow, 0)
    for cc in range(8):
        ones1d[pl.ds(cc * 16, 16)] = ones16

    def _zcnt(i, _):
        cst1d[pl.ds(i * 16, 16)] = zero16
        return 0
    lax.fori_loop(0, CNT_T // 16, _zcnt, 0)
    for k in range(4):
        acc_r = s * 512 + k * K
        pltpu.sync_copy(rows_v, acc.at[pl.ds(acc_r, K)])
    pltpu.sync_copy(cst1d, acc_cnt.at[pl.ds(s * CNT_T, CNT_T)])
    plsc.subcore_barrier()

    def _block(bi, _):
        boff = ebase + bi * (CPB * K)
        pltpu.sync_copy(cmb_hbm.at[pl.ds(boff, CPB * K)], cmb1d)
        pltpu.sync_copy(enb_hbm.at[pl.ds(boff, CPB * K)], nbr1d)

        bufs = (rows_v, rows_w)
        sems = (sem, sem2)
        descs = [None, None]
        descs[0] = pltpu.async_copy(
            nf_hbm.at[nbr1d.at[pl.ds(0, K)]], bufs[0], sems[0])
        descs[1] = pltpu.async_copy(
            nf_hbm.at[nbr1d.at[pl.ds(K, K)]], bufs[1], sems[1])
        for j in range(CPB):
            base = j * K
            for cc in range(8):
                off = base + cc * 16
                cmb = cmb1d[pl.ds(off, 16)]
                bat = lax.shift_right_logical(cmb, 4)
                rel = cmb & _i32(15)
                v2 = (rel >= 1) & (rel <= 6)
                sidx2d[j, pl.ds(cc * 16, 16)] = bat + jnp.where(
                    v2, _i32(B), _i32(0))
                cidx2d[j, pl.ds(cc * 16, 16)] = bat * _i32(NREL) + rel
            b = j % 2
            descs[b].wait()
            pltpu.sync_copy(bufs[b], acc.at[sidx2d.at[j]], add=True)
            pltpu.sync_copy(ones1d, acc_cnt.at[cidx2d.at[j]], add=True)
            if j + 2 < CPB:
                descs[b] = pltpu.async_copy(
                    nf_hbm.at[nbr1d.at[pl.ds((j + 2) * K, K)]],
                    bufs[b], sems[b])
        return 0
    lax.fori_loop(0, NBLK, _block, 0)

    for k in range(2):
        fbase = s * 256 + k * K
        pltpu.sync_copy(bni_hbm.at[pl.ds(fbase, K)], bid_c)
        pltpu.async_copy(nf_hbm.at[bid_c], rows_v, sem).wait()
        pltpu.sync_copy(rows_v, out_fb.at[pl.ds(fbase, K)])

    plsc.subcore_barrier()
    for k in range(4):
        acc_r = s * 512 + k * K
        pltpu.sync_copy(acc.at[pl.ds(acc_r, K)], rows_v)
        pltpu.sync_copy(rows_v, out_acc.at[pl.ds(acc_r, K)])
    pltpu.sync_copy(acc_cnt.at[pl.ds(s * CNT_T, CNT_T)], cst1d)
    pltpu.sync_copy(cst1d, out_cnt.at[pl.ds(s * CNT_T, CNT_T)])


_sc_agg = pl.kernel(
    _sc_body,
    out_type=(
        jax.ShapeDtypeStruct((2 * B, D), _f32),
        jax.ShapeDtypeStruct((B * NREL,), _f32),
        jax.ShapeDtypeStruct((B, D), _f32),
    ),
    mesh=plsc.VectorSubcoreMesh(
        core_axis_name="c", subcore_axis_name="s",
        num_cores=NC, num_subcores=NS),
    scratch_types=(
        pltpu.VMEM((CPB * K,), _i32),
        pltpu.VMEM((CPB * K,), _i32),
        pltpu.VMEM((CPB, K), _i32),
        pltpu.VMEM((CPB, K), _i32),
        pltpu.VMEM((K,), _i32),
        pltpu.VMEM((K, D), _f32),
        pltpu.VMEM((K, D), _f32),
        pltpu.VMEM((K,), _f32),
        pltpu.VMEM((CNT_T,), _f32),
        pltpu.VMEM_SHARED((2 * B, D), _f32),
        pltpu.VMEM_SHARED((B * NREL,), _f32),
        pltpu.SemaphoreType.DMA,
        pltpu.SemaphoreType.DMA,
    ),
)


def _tc_body(accp, cntp, fb, relp, w1a, b1a, w2a, b2a, w1b, b1b, w2b, b2b,
             out):
    cnt = cntp[...]
    col = lax.broadcasted_iota(_i32, (B, NREL), 1)
    m1 = (col == 0) | (col == 7) | (col == 8)
    cnt1 = jnp.where(m1, cnt, 0.0)
    cnt2 = jnp.where(m1, 0.0, cnt)
    n1 = jnp.sum(cnt1, axis=1, keepdims=True)
    n2 = jnp.sum(cnt2, axis=1, keepdims=True)

    rel = relp[...]
    s1 = accp[0:B] + jnp.dot(cnt1, rel, preferred_element_type=_f32)
    s2 = accp[B:2 * B] + jnp.dot(cnt2, rel, preferred_element_type=_f32)
    fbv = fb[...]
    a1 = jnp.where(n1 > 0, s1 / jnp.maximum(n1, 1.0), fbv)
    a2 = jnp.where(n2 > 0, s2 / jnp.maximum(n2, 1.0), fbv)

    def proj(x, w1, b1, w2, b2):
        h = jnp.maximum(jnp.dot(x, w1[...], preferred_element_type=_f32)
                        + b1[...], 0.0)
        return jnp.dot(h, w2[...], preferred_element_type=_f32) + b2[...]

    def l2n(z):
        nrm = jnp.sqrt(jnp.sum(z * z, axis=1, keepdims=True))
        return z / jnp.maximum(nrm, 1e-12)

    z1 = l2n(proj(a1, w1a, b1a, w2a, b2a))
    z2 = l2n(proj(a2, w1b, b1b, w2b, b2b))

    inv_t = _f32(1.0 / TEMP)
    CB = 512
    m = jnp.full((B, 1), -1e30, _f32)
    ssum = jnp.zeros((B, 1), _f32)
    col_acc = _f32(0.0)
    diag_acc = _f32(0.0)
    for j in range(B // CB):
        z2b = z2[j * CB:(j + 1) * CB]
        simb = lax.dot_general(z1, z2b, (((1,), (1,)), ((), ())),
                               preferred_element_type=_f32) * inv_t
        bm = jnp.max(simb, axis=1, keepdims=True)
        mn = jnp.maximum(m, bm)
        ssum = ssum * jnp.exp(m - mn) + jnp.sum(
            jnp.exp(simb - mn), axis=1, keepdims=True)
        m = mn
        cm = jnp.max(simb, axis=0, keepdims=True)
        col_acc = col_acc + jnp.sum(
            cm + jnp.log(jnp.sum(jnp.exp(simb - cm), axis=0, keepdims=True)))
        ri = lax.broadcasted_iota(_i32, (B, CB), 0)
        ci = lax.broadcasted_iota(_i32, (B, CB), 1) + j * CB
        diag_acc = diag_acc + jnp.sum(jnp.where(ri == ci, simb, 0.0))
    row_acc = jnp.sum(m + jnp.log(ssum))
    loss = -diag_acc / B + (row_acc + col_acc) / (2.0 * B)
    out[...] = loss.reshape(1, 1)


_tc_loss = pl.pallas_call(
    _tc_body,
    out_shape=jax.ShapeDtypeStruct((1, 1), _f32),
)


def kernel(node_features, rel_emb, W1a, b1a, W2a, b2a, W1b, b1b, W2b, b2b,
           edge_batch_idx, edge_neighbor, edge_rel, batch_node_ids):
    ebi = edge_batch_idx.astype(_i32)
    enb = edge_neighbor.astype(_i32)
    erl = edge_rel.astype(_i32)
    bni = batch_node_ids.astype(_i32)
    cmb = ebi * 16 + erl
    out_acc, out_cnt, fbv = _sc_agg(node_features, cmb, enb, bni)
    cnt = out_cnt.reshape(B, NREL)
    loss = _tc_loss(out_acc, cnt, fbv, rel_emb,
                    W1a, b1a.reshape(1, D), W2a, b2a.reshape(1, HALF),
                    W1b, b1b.reshape(1, D), W2b, b2b.reshape(1, HALF))
    return loss.reshape(())


# speedup vs baseline: 14.4173x; 1.0699x over previous
import jax
import jax.numpy as jnp
from jax import lax
from jax.experimental import pallas as pl
from jax.experimental.pallas import tpu as pltpu
from jax.experimental.pallas import tpu_sc as plsc

D = 128
DH = 64
HALF = 64
B = 4096
E = 262144
NREL = 9
TEMP = 0.07

NC = 1
NS = 16
EPT = E // NS
K = 128
NCHUNK = EPT // K
NBLK = 8
CPB = NCHUNK // NBLK
CNT_T = B * NREL // NS

_f32 = jnp.float32
_i32 = jnp.int32


def _sc_body(nf_hbm, cmb_hbm, enb_hbm, bni_hbm, out_acc, out_cnt, out_fb,
             nbr1d, cmb1d, sidx2d, cidx2d, bid_c, rows_v, rows_w, ones1d,
             cst1d, acc, acc_cnt, sem, sem2):
    s = lax.axis_index("s")
    ebase = s * EPT

    zero16 = jnp.zeros((16,), _f32)

    ones16 = jnp.ones((16,), _f32)

    def _zrow(i, _):
        for cc in range(8):
            rows_v[i, pl.ds(cc * 16, 16)] = zero16
        return 0
    lax.fori_loop(0, K, _zrow, 0)
    for cc in range(8):
        ones1d[pl.ds(cc * 16, 16)] = ones16

    def _zcnt(i, _):
        cst1d[pl.ds(i * 16, 16)] = zero16
        return 0
    lax.fori_loop(0, CNT_T // 16, _zcnt, 0)
    for k in range(4):
        acc_r = s * 512 + k * K
        pltpu.sync_copy(rows_v, acc.at[pl.ds(acc_r, K)])
    pltpu.sync_copy(cst1d, acc_cnt.at[pl.ds(s * CNT_T, CNT_T)])
    plsc.subcore_barrier()

    def _block(bi, _):
        boff = ebase + bi * (CPB * K)
        pltpu.sync_copy(cmb_hbm.at[pl.ds(boff, CPB * K)], cmb1d)
        pltpu.sync_copy(enb_hbm.at[pl.ds(boff, CPB * K)], nbr1d)

        bufs = (rows_v, rows_w)
        sems = (sem, sem2)
        descs = [None, None]
        descs[0] = pltpu.async_copy(
            nf_hbm.at[nbr1d.at[pl.ds(0, K)]], bufs[0], sems[0])
        descs[1] = pltpu.async_copy(
            nf_hbm.at[nbr1d.at[pl.ds(K, K)]], bufs[1], sems[1])
        for j in range(CPB):
            base = j * K
            for cc in range(8):
                off = base + cc * 16
                cmb = cmb1d[pl.ds(off, 16)]
                bat = lax.shift_right_logical(cmb, 4)
                rel = cmb & _i32(15)
                v2 = (rel >= 1) & (rel <= 6)
                sidx2d[j, pl.ds(cc * 16, 16)] = bat + jnp.where(
                    v2, _i32(B), _i32(0))
                cidx2d[j, pl.ds(cc * 16, 16)] = bat * _i32(NREL) + rel
            b = j % 2
            descs[b].wait()
            pltpu.sync_copy(bufs[b], acc.at[sidx2d.at[j]], add=True)
            pltpu.sync_copy(ones1d, acc_cnt.at[cidx2d.at[j]], add=True)
            if j + 2 < CPB:
                descs[b] = pltpu.async_copy(
                    nf_hbm.at[nbr1d.at[pl.ds((j + 2) * K, K)]],
                    bufs[b], sems[b])
        return 0
    lax.fori_loop(0, NBLK, _block, 0)

    for k in range(2):
        fbase = s * 256 + k * K
        pltpu.sync_copy(bni_hbm.at[pl.ds(fbase, K)], bid_c)
        pltpu.async_copy(nf_hbm.at[bid_c], rows_v, sem).wait()
        pltpu.sync_copy(rows_v, out_fb.at[pl.ds(fbase, K)])

    plsc.subcore_barrier()
    for k in range(4):
        acc_r = s * 512 + k * K
        pltpu.sync_copy(acc.at[pl.ds(acc_r, K)], rows_v)
        pltpu.sync_copy(rows_v, out_acc.at[pl.ds(acc_r, K)])
    pltpu.sync_copy(acc_cnt.at[pl.ds(s * CNT_T, CNT_T)], cst1d)
    pltpu.sync_copy(cst1d, out_cnt.at[pl.ds(s * CNT_T, CNT_T)])


_sc_agg = pl.kernel(
    _sc_body,
    out_type=(
        jax.ShapeDtypeStruct((2 * B, D), _f32),
        jax.ShapeDtypeStruct((B * NREL,), _f32),
        jax.ShapeDtypeStruct((B, D), _f32),
    ),
    mesh=plsc.VectorSubcoreMesh(
        core_axis_name="c", subcore_axis_name="s",
        num_cores=NC, num_subcores=NS),
    scratch_types=(
        pltpu.VMEM((CPB * K,), _i32),
        pltpu.VMEM((CPB * K,), _i32),
        pltpu.VMEM((CPB, K), _i32),
        pltpu.VMEM((CPB, K), _i32),
        pltpu.VMEM((K,), _i32),
        pltpu.VMEM((K, D), _f32),
        pltpu.VMEM((K, D), _f32),
        pltpu.VMEM((K,), _f32),
        pltpu.VMEM((CNT_T,), _f32),
        pltpu.VMEM_SHARED((2 * B, D), _f32),
        pltpu.VMEM_SHARED((B * NREL,), _f32),
        pltpu.SemaphoreType.DMA,
        pltpu.SemaphoreType.DMA,
    ),
)


def _tc_body(accp, cntp, fb, relp, w1a, b1a, w2a, b2a, w1b, b1b, w2b, b2b,
             out):
    cnt = cntp[...]
    col = lax.broadcasted_iota(_i32, (B, NREL), 1)
    m1 = (col == 0) | (col == 7) | (col == 8)
    cnt1 = jnp.where(m1, cnt, 0.0)
    cnt2 = jnp.where(m1, 0.0, cnt)
    n1 = jnp.sum(cnt1, axis=1, keepdims=True)
    n2 = jnp.sum(cnt2, axis=1, keepdims=True)

    rel = relp[...]
    s1 = accp[0:B] + jnp.dot(cnt1, rel, preferred_element_type=_f32)
    s2 = accp[B:2 * B] + jnp.dot(cnt2, rel, preferred_element_type=_f32)
    fbv = fb[...]
    a1 = jnp.where(n1 > 0, s1 / jnp.maximum(n1, 1.0), fbv)
    a2 = jnp.where(n2 > 0, s2 / jnp.maximum(n2, 1.0), fbv)

    def proj(x, w1, b1, w2, b2):
        h = jnp.maximum(jnp.dot(x, w1[...], preferred_element_type=_f32)
                        + b1[...], 0.0)
        return jnp.dot(h, w2[...], preferred_element_type=_f32) + b2[...]

    def l2n(z):
        nrm = jnp.sqrt(jnp.sum(z * z, axis=1, keepdims=True))
        return z / jnp.maximum(nrm, 1e-12)

    z1 = l2n(proj(a1, w1a, b1a, w2a, b2a))
    z2 = l2n(proj(a2, w1b, b1b, w2b, b2b))

    inv_t = _f32(1.0 / TEMP)
    CB = 512
    m = jnp.full((B, 1), -1e30, _f32)
    ssum = jnp.zeros((B, 1), _f32)
    col_acc = _f32(0.0)
    diag_acc = _f32(0.0)
    for j in range(B // CB):
        z2b = z2[j * CB:(j + 1) * CB]
        simb = lax.dot_general(z1, z2b, (((1,), (1,)), ((), ())),
                               preferred_element_type=_f32) * inv_t
        bm = jnp.max(simb, axis=1, keepdims=True)
        mn = jnp.maximum(m, bm)
        ssum = ssum * jnp.exp(m - mn) + jnp.sum(
            jnp.exp(simb - mn), axis=1, keepdims=True)
        m = mn
        cm = jnp.max(simb, axis=0, keepdims=True)
        col_acc = col_acc + jnp.sum(
            cm + jnp.log(jnp.sum(jnp.exp(simb - cm), axis=0, keepdims=True)))
        ri = lax.broadcasted_iota(_i32, (B, CB), 0)
        ci = lax.broadcasted_iota(_i32, (B, CB), 1) + j * CB
        diag_acc = diag_acc + jnp.sum(jnp.where(ri == ci, simb, 0.0))
    row_acc = jnp.sum(m + jnp.log(ssum))
    loss = -diag_acc / B + (row_acc + col_acc) / (2.0 * B)
    out[...] = loss.reshape(1, 1)


_tc_loss = pl.pallas_call(
    _tc_body,
    out_shape=jax.ShapeDtypeStruct((1, 1), _f32),
)


def kernel(node_features, rel_emb, W1a, b1a, W2a, b2a, W1b, b1b, W2b, b2b,
           edge_batch_idx, edge_neighbor, edge_rel, batch_node_ids):
    ebi = edge_batch_idx.astype(_i32)
    enb = edge_neighbor.astype(_i32)
    erl = edge_rel.astype(_i32)
    bni = batch_node_ids.astype(_i32)
    cmb = ebi * 16 + erl
    out_acc, out_cnt, fbv = _sc_agg(node_features, cmb, enb, bni)
    cnt = out_cnt.reshape(B, NREL)
    loss = _tc_loss(out_acc, cnt, fbv, rel_emb,
                    W1a, b1a.reshape(1, D), W2a, b2a.reshape(1, HALF),
                    W1b, b1b.reshape(1, D), W2b, b2b.reshape(1, HALF))
    return loss.reshape(())


# speedup vs baseline: 15.7885x; 1.0951x over previous
import jax
import jax.numpy as jnp
from jax import lax
from jax.experimental import pallas as pl
from jax.experimental.pallas import tpu as pltpu
from jax.experimental.pallas import tpu_sc as plsc

D = 128
DH = 64
HALF = 64
B = 4096
E = 262144
NREL = 9
TEMP = 0.07

NC = 1
NS = 16
EPT = E // NS
K = 128
NCHUNK = EPT // K
NBLK = 8
CPB = NCHUNK // NBLK
CNT_T = B * NREL // NS

_f32 = jnp.float32
_i32 = jnp.int32


def _sc_body(nf_hbm, cmb_hbm, enb_hbm, bni_hbm, out_acc, out_cnt, out_fb,
             nbr1d, cmb1d, sidx2d, cidx2d, bid_c, rows_v, rows_w, ones1d,
             cst1d, acc, acc_cnt, sem, sem2):
    s = lax.axis_index("s")
    ebase = s * EPT

    zero16 = jnp.zeros((16,), _f32)

    ones16 = jnp.ones((16,), _f32)

    def _zrow(i, _):
        for cc in range(8):
            rows_v[i, pl.ds(cc * 16, 16)] = zero16
        return 0
    lax.fori_loop(0, K, _zrow, 0)
    for cc in range(8):
        ones1d[pl.ds(cc * 16, 16)] = ones16

    def _zcnt(i, _):
        cst1d[pl.ds(i * 16, 16)] = zero16
        return 0
    lax.fori_loop(0, CNT_T // 16, _zcnt, 0)
    for k in range(4):
        acc_r = s * 512 + k * K
        pltpu.sync_copy(rows_v, acc.at[pl.ds(acc_r, K)])
    pltpu.sync_copy(cst1d, acc_cnt.at[pl.ds(s * CNT_T, CNT_T)])
    plsc.subcore_barrier()

    def _block(bi, _):
        boff = ebase + bi * (CPB * K)
        pltpu.sync_copy(cmb_hbm.at[pl.ds(boff, CPB * K)], cmb1d)
        pltpu.sync_copy(enb_hbm.at[pl.ds(boff, CPB * K)], nbr1d)

        bufs = (rows_v, rows_w)
        sems = (sem, sem2)
        descs = [None, None]
        descs[0] = pltpu.async_copy(
            nf_hbm.at[nbr1d.at[pl.ds(0, K)]], bufs[0], sems[0])
        descs[1] = pltpu.async_copy(
            nf_hbm.at[nbr1d.at[pl.ds(K, K)]], bufs[1], sems[1])
        for j in range(CPB):
            base = j * K
            for cc in range(8):
                off = base + cc * 16
                cmb = cmb1d[pl.ds(off, 16)]
                bat = lax.shift_right_logical(cmb, 4)
                rel = cmb & _i32(15)
                v2 = (rel >= 1) & (rel <= 6)
                sidx2d[j, pl.ds(cc * 16, 16)] = bat + jnp.where(
                    v2, _i32(B), _i32(0))
                cidx2d[j, pl.ds(cc * 16, 16)] = bat * _i32(NREL) + rel
            b = j % 2
            descs[b].wait()
            pltpu.sync_copy(bufs[b], acc.at[sidx2d.at[j]], add=True)
            pltpu.sync_copy(ones1d, acc_cnt.at[cidx2d.at[j]], add=True)
            if j + 2 < CPB:
                descs[b] = pltpu.async_copy(
                    nf_hbm.at[nbr1d.at[pl.ds((j + 2) * K, K)]],
                    bufs[b], sems[b])
        return 0
    lax.fori_loop(0, NBLK, _block, 0)

    for k in range(2):
        fbase = s * 256 + k * K
        pltpu.sync_copy(bni_hbm.at[pl.ds(fbase, K)], bid_c)
        pltpu.async_copy(nf_hbm.at[bid_c], rows_v, sem).wait()
        pltpu.sync_copy(rows_v, out_fb.at[pl.ds(fbase, K)])

    plsc.subcore_barrier()
    for k in range(4):
        acc_r = s * 512 + k * K
        pltpu.sync_copy(acc.at[pl.ds(acc_r, K)], rows_v)
        pltpu.sync_copy(rows_v, out_acc.at[pl.ds(acc_r, K)])
    pltpu.sync_copy(acc_cnt.at[pl.ds(s * CNT_T, CNT_T)], cst1d)
    pltpu.sync_copy(cst1d, out_cnt.at[pl.ds(s * CNT_T, CNT_T)])


_sc_agg = pl.kernel(
    _sc_body,
    out_type=(
        jax.ShapeDtypeStruct((2 * B, D), _f32),
        jax.ShapeDtypeStruct((B * NREL,), _f32),
        jax.ShapeDtypeStruct((B, D), _f32),
    ),
    mesh=plsc.VectorSubcoreMesh(
        core_axis_name="c", subcore_axis_name="s",
        num_cores=NC, num_subcores=NS),
    scratch_types=(
        pltpu.VMEM((CPB * K,), _i32),
        pltpu.VMEM((CPB * K,), _i32),
        pltpu.VMEM((CPB, K), _i32),
        pltpu.VMEM((CPB, K), _i32),
        pltpu.VMEM((K,), _i32),
        pltpu.VMEM((K, D), _f32),
        pltpu.VMEM((K, D), _f32),
        pltpu.VMEM((K,), _f32),
        pltpu.VMEM((CNT_T,), _f32),
        pltpu.VMEM_SHARED((2 * B, D), _f32),
        pltpu.VMEM_SHARED((B * NREL,), _f32),
        pltpu.SemaphoreType.DMA,
        pltpu.SemaphoreType.DMA,
    ),
)


def _tc_body(accp, cntp, fb, relp, w1a, b1a, w2a, b2a, w1b, b1b, w2b, b2b,
             out):
    cnt = cntp[...]
    col = lax.broadcasted_iota(_i32, (B, NREL), 1)
    m1 = (col == 0) | (col == 7) | (col == 8)
    cnt1 = jnp.where(m1, cnt, 0.0)
    cnt2 = jnp.where(m1, 0.0, cnt)
    n1 = jnp.sum(cnt1, axis=1, keepdims=True)
    n2 = jnp.sum(cnt2, axis=1, keepdims=True)

    rel = relp[...]
    s1 = accp[0:B] + jnp.dot(cnt1, rel, preferred_element_type=_f32)
    s2 = accp[B:2 * B] + jnp.dot(cnt2, rel, preferred_element_type=_f32)
    fbv = fb[...]
    a1 = jnp.where(n1 > 0, s1 / jnp.maximum(n1, 1.0), fbv)
    a2 = jnp.where(n2 > 0, s2 / jnp.maximum(n2, 1.0), fbv)

    def proj(x, w1, b1, w2, b2):
        h = jnp.maximum(jnp.dot(x, w1[...], preferred_element_type=_f32)
                        + b1[...], 0.0)
        return jnp.dot(h, w2[...], preferred_element_type=_f32) + b2[...]

    def l2n(z):
        nrm = jnp.sqrt(jnp.sum(z * z, axis=1, keepdims=True))
        return z / jnp.maximum(nrm, 1e-12)

    z1 = l2n(proj(a1, w1a, b1a, w2a, b2a))
    z2 = l2n(proj(a2, w1b, b1b, w2b, b2b))

    inv_t = _f32(1.0 / TEMP)
    CB = 1024
    diag_acc = jnp.sum(z1 * z2) * inv_t
    ssum = jnp.zeros((B, 1), _f32)
    col_acc = _f32(0.0)
    for j in range(B // CB):
        z2b = z2[j * CB:(j + 1) * CB]
        simb = lax.dot_general(z1, z2b, (((1,), (1,)), ((), ())),
                               preferred_element_type=_f32) * inv_t
        p = jnp.exp(simb)
        ssum = ssum + jnp.sum(p, axis=1, keepdims=True)
        col_acc = col_acc + jnp.sum(
            jnp.log(jnp.sum(p, axis=0, keepdims=True)))
    row_acc = jnp.sum(jnp.log(ssum))
    loss = -diag_acc / B + (row_acc + col_acc) / (2.0 * B)
    out[...] = loss.reshape(1, 1)


_tc_loss = pl.pallas_call(
    _tc_body,
    out_shape=jax.ShapeDtypeStruct((1, 1), _f32),
)


def kernel(node_features, rel_emb, W1a, b1a, W2a, b2a, W1b, b1b, W2b, b2b,
           edge_batch_idx, edge_neighbor, edge_rel, batch_node_ids):
    ebi = edge_batch_idx.astype(_i32)
    enb = edge_neighbor.astype(_i32)
    erl = edge_rel.astype(_i32)
    bni = batch_node_ids.astype(_i32)
    cmb = ebi * 16 + erl
    out_acc, out_cnt, fbv = _sc_agg(node_features, cmb, enb, bni)
    cnt = out_cnt.reshape(B, NREL)
    loss = _tc_loss(out_acc, cnt, fbv, rel_emb,
                    W1a, b1a.reshape(1, D), W2a, b2a.reshape(1, HALF),
                    W1b, b1b.reshape(1, D), W2b, b2b.reshape(1, HALF))
    return loss.reshape(())


# speedup vs baseline: 16.1341x; 1.0219x over previous
import jax
import jax.numpy as jnp
from jax import lax
from jax.experimental import pallas as pl
from jax.experimental.pallas import tpu as pltpu
from jax.experimental.pallas import tpu_sc as plsc

D = 128
DH = 64
HALF = 64
B = 4096
E = 262144
NREL = 9
TEMP = 0.07

NC = 1
NS = 16
EPT = E // NS
K = 128
NCHUNK = EPT // K
NBLK = 8
CPB = NCHUNK // NBLK
CNT_T = B * NREL // NS

_f32 = jnp.float32
_i32 = jnp.int32


def _sc_body(nf_hbm, cmb_hbm, enb_hbm, bni_hbm, out_acc, out_cnt, out_fb,
             nbr1d, cmb1d, sidx2d, cidx2d, bid_c, rows_v, rows_w, rows_x,
             ones1d, cst1d, acc, acc_cnt, sem, sem2, sem3, sem_s, sem_c):
    s = lax.axis_index("s")
    ebase = s * EPT

    zero16 = jnp.zeros((16,), _f32)

    ones16 = jnp.ones((16,), _f32)

    def _zrow(i, _):
        for cc in range(8):
            rows_v[i, pl.ds(cc * 16, 16)] = zero16
        return 0
    lax.fori_loop(0, K, _zrow, 0)
    for cc in range(8):
        ones1d[pl.ds(cc * 16, 16)] = ones16

    def _zcnt(i, _):
        cst1d[pl.ds(i * 16, 16)] = zero16
        return 0
    lax.fori_loop(0, CNT_T // 16, _zcnt, 0)
    for k in range(4):
        acc_r = s * 512 + k * K
        pltpu.sync_copy(rows_v, acc.at[pl.ds(acc_r, K)])
    pltpu.sync_copy(cst1d, acc_cnt.at[pl.ds(s * CNT_T, CNT_T)])
    plsc.subcore_barrier()

    def _block(bi, _):
        boff = ebase + bi * (CPB * K)
        pltpu.sync_copy(cmb_hbm.at[pl.ds(boff, CPB * K)], cmb1d)
        pltpu.sync_copy(enb_hbm.at[pl.ds(boff, CPB * K)], nbr1d)

        bufs = (rows_v, rows_w, rows_x)
        sems = (sem, sem2, sem3)
        gd = [None, None, None]
        sd = [None] * CPB
        cd = [None] * CPB
        gd[0] = pltpu.async_copy(
            nf_hbm.at[nbr1d.at[pl.ds(0, K)]], bufs[0], sems[0])
        gd[1] = pltpu.async_copy(
            nf_hbm.at[nbr1d.at[pl.ds(K, K)]], bufs[1], sems[1])
        gd[2] = pltpu.async_copy(
            nf_hbm.at[nbr1d.at[pl.ds(2 * K, K)]], bufs[2], sems[2])
        for j in range(CPB):
            base = j * K
            for cc in range(8):
                off = base + cc * 16
                cmb = cmb1d[pl.ds(off, 16)]
                bat = lax.shift_right_logical(cmb, 4)
                rel = cmb & _i32(15)
                v2 = (rel >= 1) & (rel <= 6)
                sidx2d[j, pl.ds(cc * 16, 16)] = bat + jnp.where(
                    v2, _i32(B), _i32(0))
                cidx2d[j, pl.ds(cc * 16, 16)] = bat * _i32(NREL) + rel
            b = j % 3
            gd[b].wait()
            sd[j] = pltpu.async_copy(bufs[b], acc.at[sidx2d.at[j]], sem_s,
                                     add=True)
            cd[j] = pltpu.async_copy(ones1d, acc_cnt.at[cidx2d.at[j]],
                                     sem_c, add=True)
            if j + 3 < CPB:
                sd[j].wait()
                gd[b] = pltpu.async_copy(
                    nf_hbm.at[nbr1d.at[pl.ds((j + 3) * K, K)]],
                    bufs[b], sems[b])
        for j in range(CPB - 3, CPB):
            sd[j].wait()
        for j in range(CPB):
            cd[j].wait()
        return 0
    lax.fori_loop(0, NBLK, _block, 0)

    for k in range(2):
        fbase = s * 256 + k * K
        pltpu.sync_copy(bni_hbm.at[pl.ds(fbase, K)], bid_c)
        pltpu.async_copy(nf_hbm.at[bid_c], rows_v, sem).wait()
        pltpu.sync_copy(rows_v, out_fb.at[pl.ds(fbase, K)])

    plsc.subcore_barrier()
    for k in range(4):
        acc_r = s * 512 + k * K
        pltpu.sync_copy(acc.at[pl.ds(acc_r, K)], rows_v)
        pltpu.sync_copy(rows_v, out_acc.at[pl.ds(acc_r, K)])
    pltpu.sync_copy(acc_cnt.at[pl.ds(s * CNT_T, CNT_T)], cst1d)
    pltpu.sync_copy(cst1d, out_cnt.at[pl.ds(s * CNT_T, CNT_T)])


_sc_agg = pl.kernel(
    _sc_body,
    out_type=(
        jax.ShapeDtypeStruct((2 * B, D), _f32),
        jax.ShapeDtypeStruct((B * NREL,), _f32),
        jax.ShapeDtypeStruct((B, D), _f32),
    ),
    mesh=plsc.VectorSubcoreMesh(
        core_axis_name="c", subcore_axis_name="s",
        num_cores=NC, num_subcores=NS),
    scratch_types=(
        pltpu.VMEM((CPB * K,), _i32),
        pltpu.VMEM((CPB * K,), _i32),
        pltpu.VMEM((CPB, K), _i32),
        pltpu.VMEM((CPB, K), _i32),
        pltpu.VMEM((K,), _i32),
        pltpu.VMEM((K, D), _f32),
        pltpu.VMEM((K, D), _f32),
        pltpu.VMEM((K, D), _f32),
        pltpu.VMEM((K,), _f32),
        pltpu.VMEM((CNT_T,), _f32),
        pltpu.VMEM_SHARED((2 * B, D), _f32),
        pltpu.VMEM_SHARED((B * NREL,), _f32),
        pltpu.SemaphoreType.DMA,
        pltpu.SemaphoreType.DMA,
        pltpu.SemaphoreType.DMA,
        pltpu.SemaphoreType.DMA,
        pltpu.SemaphoreType.DMA,
    ),
)


def _tc_body(accp, cntp, fb, relp, w1a, b1a, w2a, b2a, w1b, b1b, w2b, b2b,
             out):
    cnt = cntp[...]
    col = lax.broadcasted_iota(_i32, (B, NREL), 1)
    m1 = (col == 0) | (col == 7) | (col == 8)
    cnt1 = jnp.where(m1, cnt, 0.0)
    cnt2 = jnp.where(m1, 0.0, cnt)
    n1 = jnp.sum(cnt1, axis=1, keepdims=True)
    n2 = jnp.sum(cnt2, axis=1, keepdims=True)

    rel = relp[...]
    s1 = accp[0:B] + jnp.dot(cnt1, rel, preferred_element_type=_f32)
    s2 = accp[B:2 * B] + jnp.dot(cnt2, rel, preferred_element_type=_f32)
    fbv = fb[...]
    a1 = jnp.where(n1 > 0, s1 / jnp.maximum(n1, 1.0), fbv)
    a2 = jnp.where(n2 > 0, s2 / jnp.maximum(n2, 1.0), fbv)

    def proj(x, w1, b1, w2, b2):
        h = jnp.maximum(jnp.dot(x, w1[...], preferred_element_type=_f32)
                        + b1[...], 0.0)
        return jnp.dot(h, w2[...], preferred_element_type=_f32) + b2[...]

    def l2n(z):
        nrm = jnp.sqrt(jnp.sum(z * z, axis=1, keepdims=True))
        return z / jnp.maximum(nrm, 1e-12)

    z1 = l2n(proj(a1, w1a, b1a, w2a, b2a))
    z2 = l2n(proj(a2, w1b, b1b, w2b, b2b))

    inv_t = _f32(1.0 / TEMP)
    CB = 1024
    diag_acc = jnp.sum(z1 * z2) * inv_t
    ssum = jnp.zeros((B, 1), _f32)
    col_acc = _f32(0.0)
    for j in range(B // CB):
        z2b = z2[j * CB:(j + 1) * CB]
        simb = lax.dot_general(z1, z2b, (((1,), (1,)), ((), ())),
                               preferred_element_type=_f32) * inv_t
        p = jnp.exp(simb)
        ssum = ssum + jnp.sum(p, axis=1, keepdims=True)
        col_acc = col_acc + jnp.sum(
            jnp.log(jnp.sum(p, axis=0, keepdims=True)))
    row_acc = jnp.sum(jnp.log(ssum))
    loss = -diag_acc / B + (row_acc + col_acc) / (2.0 * B)
    out[...] = loss.reshape(1, 1)


_tc_loss = pl.pallas_call(
    _tc_body,
    out_shape=jax.ShapeDtypeStruct((1, 1), _f32),
)


def kernel(node_features, rel_emb, W1a, b1a, W2a, b2a, W1b, b1b, W2b, b2b,
           edge_batch_idx, edge_neighbor, edge_rel, batch_node_ids):
    ebi = edge_batch_idx.astype(_i32)
    enb = edge_neighbor.astype(_i32)
    erl = edge_rel.astype(_i32)
    bni = batch_node_ids.astype(_i32)
    cmb = ebi * 16 + erl
    out_acc, out_cnt, fbv = _sc_agg(node_features, cmb, enb, bni)
    cnt = out_cnt.reshape(B, NREL)
    loss = _tc_loss(out_acc, cnt, fbv, rel_emb,
                    W1a, b1a.reshape(1, D), W2a, b2a.reshape(1, HALF),
                    W1b, b1b.reshape(1, D), W2b, b2b.reshape(1, HALF))
    return loss.reshape(())
